# baseline (speedup 1.0000x reference)
import functools

import jax
import jax.numpy as jnp
from jax import lax
from jax.experimental import pallas as pl
from jax.experimental.pallas import tpu as pltpu
from jax.experimental.pallas import tpu_sc as plsc

_EMB_DIM = 64
_WIDE = 128
_NC, _NS = 2, 16
_NW = _NC * _NS
_CHUNK = 128


def _transform_table(table_t, w, b):
    v = table_t.shape[1]
    v_pad = -8 * (-v // 8)
    blk = 8192
    grid = (v_pad + blk - 1) // blk
    wp = jnp.pad(w, ((0, _WIDE - _EMB_DIM), (0, 0)))
    bp = jnp.pad(b, (0, _WIDE - _EMB_DIM)).reshape(1, _WIDE)

    def body(x_ref, w_ref, b_ref, o_ref):
        o_ref[...] = lax.dot_general(
            x_ref[...], w_ref[...],
            (((0,), (1,)), ((), ())),
            preferred_element_type=jnp.float32,
        ) + b_ref[...]

    return pl.pallas_call(
        body,
        grid=(grid,),
        in_specs=[
            pl.BlockSpec((_EMB_DIM, blk), lambda i: (0, i)),
            pl.BlockSpec((_WIDE, _EMB_DIM), lambda i: (0, 0)),
            pl.BlockSpec((1, _WIDE), lambda i: (0, 0)),
        ],
        out_specs=pl.BlockSpec((blk, _WIDE), lambda i: (i, 0)),
        out_shape=jax.ShapeDtypeStruct((v_pad, _WIDE), jnp.float32),
    )(table_t, wp, bp)


def _gather_rows(table, gids, n_chunks, n_rows):
    per_w = n_chunks * _CHUNK
    mesh = plsc.VectorSubcoreMesh(core_axis_name="c", subcore_axis_name="s")

    nbuf = 4

    @functools.partial(
        pl.kernel,
        mesh=mesh,
        out_type=jax.ShapeDtypeStruct((n_rows, _WIDE), jnp.float32),
        scratch_types=[
            pltpu.VMEM((per_w,), jnp.int32),
            pltpu.VMEM((nbuf, _CHUNK, _WIDE), jnp.float32),
            pltpu.SemaphoreType.DMA,
            pltpu.SemaphoreType.DMA,
            pltpu.SemaphoreType.DMA,
            pltpu.SemaphoreType.DMA,
        ],
        compiler_params=pltpu.CompilerParams(use_tc_tiling_on_sc=False),
    )
    def k(tbl_hbm, gids_hbm, out_hbm, idx_v, rows_v, sem0, sem1, sem2, sem3):
        wid = lax.axis_index("s") * _NC + lax.axis_index("c")
        base = wid * per_w
        base_c = jnp.minimum(base, n_rows - per_w)
        pltpu.sync_copy(gids_hbm.at[pl.ds(base_c, per_w)], idx_v)
        sems = (sem0, sem1, sem2, sem3)

        def off_of(j):
            return jnp.minimum(base + j * _CHUNK, n_rows - _CHUNK)

        def gather_desc(j, buf):
            idx = idx_v.at[pl.ds(off_of(j) - base_c, _CHUNK)]
            return pltpu.make_async_copy(
                tbl_hbm.at[idx], rows_v.at[buf], sems[buf])

        def write(j, buf):
            pltpu.sync_copy(
                rows_v.at[buf], out_hbm.at[pl.ds(off_of(j), _CHUNK)])

        for j in range(min(nbuf - 1, n_chunks)):
            gather_desc(j, j).start()
        pre = (n_chunks - (nbuf - 1)) % nbuf
        for j in range(pre):
            nxt = j + nbuf - 1
            if nxt < n_chunks:
                gather_desc(nxt, nxt % nbuf).start()
            gather_desc(j, j % nbuf).wait()
            write(j, j % nbuf)
        n_main = max(n_chunks - (nbuf - 1) - pre, 0) // nbuf

        def body(g, carry):
            jb = pre + g * nbuf
            for s in range(nbuf):
                j = jb + s
                buf = (pre + s) % nbuf
                gather_desc(j + nbuf - 1, (buf + nbuf - 1) % nbuf).start()
                gather_desc(j, buf).wait()
                write(j, buf)
            return carry

        if n_main > 0:
            lax.fori_loop(0, n_main, body, 0)
        for j in range(pre + nbuf * n_main, n_chunks):
            gather_desc(j, j % nbuf).wait()
            write(j, j % nbuf)

    return k(table, gids)


def _transpose_out(wide, n_rows):
    blk = 8192
    grid = (n_rows + blk - 1) // blk
    eye = jnp.eye(_EMB_DIM, dtype=jnp.float32)

    def body(e_ref, x_ref, o_ref):
        del e_ref
        o_ref[...] = x_ref[:, : _EMB_DIM].T

    return pl.pallas_call(
        body,
        grid=(grid,),
        in_specs=[
            pl.BlockSpec((_EMB_DIM, _EMB_DIM), lambda i: (0, 0)),
            pl.BlockSpec((blk, _WIDE), lambda i: (i, 0)),
        ],
        out_specs=pl.BlockSpec((_EMB_DIM, blk), lambda i: (0, i)),
        out_shape=jax.ShapeDtypeStruct((_EMB_DIM, n_rows), jnp.float32),
    )(eye, wide)


def kernel(ids, adj, layer_idx, emb_table, fc_w, fc_b):
    del adj
    n_nodes = emb_table.shape[0] - 1
    b = ids.shape[0]
    t = _transform_table(emb_table.T, fc_w, fc_b)
    gids = jnp.where(layer_idx > 0, ids, n_nodes).astype(jnp.int32)
    n_chunks = -(-b // (_NW * _CHUNK))
    wide = _gather_rows(t, gids, n_chunks, b)
    return _transpose_out(wide, b).T


# speedup vs baseline: 1.0529x; 1.0529x over previous
import functools

import jax
import jax.numpy as jnp
from jax import lax
from jax.experimental import pallas as pl
from jax.experimental.pallas import tpu as pltpu
from jax.experimental.pallas import tpu_sc as plsc

_EMB_DIM = 64
_WIDE = 128
_NC, _NS = 2, 16
_NW = _NC * _NS
_CHUNK = 128


def _transform_table(table_t, w, b):
    v = table_t.shape[1]
    v_pad = -8 * (-v // 8)
    blk = 16384
    grid = (v_pad + blk - 1) // blk
    wp = jnp.pad(w, ((0, _WIDE - _EMB_DIM), (0, 0)))
    bp = jnp.pad(b, (0, _WIDE - _EMB_DIM)).reshape(1, _WIDE)

    def body(x_ref, w_ref, b_ref, o_ref):
        o_ref[...] = lax.dot_general(
            x_ref[...], w_ref[...],
            (((0,), (1,)), ((), ())),
            preferred_element_type=jnp.float32,
        ) + b_ref[...]

    return pl.pallas_call(
        body,
        grid=(grid,),
        in_specs=[
            pl.BlockSpec((_EMB_DIM, blk), lambda i: (0, i)),
            pl.BlockSpec((_WIDE, _EMB_DIM), lambda i: (0, 0)),
            pl.BlockSpec((1, _WIDE), lambda i: (0, 0)),
        ],
        out_specs=pl.BlockSpec((blk, _WIDE), lambda i: (i, 0)),
        out_shape=jax.ShapeDtypeStruct((v_pad, _WIDE), jnp.float32),
    )(table_t, wp, bp)


def _gather_rows(table, gids, n_chunks, n_rows):
    per_w = n_chunks * _CHUNK
    mesh = plsc.VectorSubcoreMesh(core_axis_name="c", subcore_axis_name="s")

    nbuf = 4

    @functools.partial(
        pl.kernel,
        mesh=mesh,
        out_type=jax.ShapeDtypeStruct((n_rows, _WIDE), jnp.float32),
        scratch_types=[
            pltpu.VMEM((per_w,), jnp.int32),
            pltpu.VMEM((nbuf, _CHUNK, _WIDE), jnp.float32),
            pltpu.SemaphoreType.DMA,
            pltpu.SemaphoreType.DMA,
            pltpu.SemaphoreType.DMA,
            pltpu.SemaphoreType.DMA,
        ],
        compiler_params=pltpu.CompilerParams(use_tc_tiling_on_sc=False),
    )
    def k(tbl_hbm, gids_hbm, out_hbm, idx_v, rows_v, sem0, sem1, sem2, sem3):
        wid = lax.axis_index("s") * _NC + lax.axis_index("c")
        base = wid * per_w
        base_c = jnp.minimum(base, n_rows - per_w)
        pltpu.sync_copy(gids_hbm.at[pl.ds(base_c, per_w)], idx_v)
        sems = (sem0, sem1, sem2, sem3)

        def off_of(j):
            return jnp.minimum(base + j * _CHUNK, n_rows - _CHUNK)

        def gather_desc(j, buf):
            idx = idx_v.at[pl.ds(off_of(j) - base_c, _CHUNK)]
            return pltpu.make_async_copy(
                tbl_hbm.at[idx], rows_v.at[buf], sems[buf])

        def write(j, buf):
            pltpu.sync_copy(
                rows_v.at[buf], out_hbm.at[pl.ds(off_of(j), _CHUNK)])

        for j in range(min(nbuf - 1, n_chunks)):
            gather_desc(j, j).start()
        pre = (n_chunks - (nbuf - 1)) % nbuf
        for j in range(pre):
            nxt = j + nbuf - 1
            if nxt < n_chunks:
                gather_desc(nxt, nxt % nbuf).start()
            gather_desc(j, j % nbuf).wait()
            write(j, j % nbuf)
        n_main = max(n_chunks - (nbuf - 1) - pre, 0) // nbuf

        def body(g, carry):
            jb = pre + g * nbuf
            for s in range(nbuf):
                j = jb + s
                buf = (pre + s) % nbuf
                gather_desc(j + nbuf - 1, (buf + nbuf - 1) % nbuf).start()
                gather_desc(j, buf).wait()
                write(j, buf)
            return carry

        if n_main > 0:
            lax.fori_loop(0, n_main, body, 0)
        for j in range(pre + nbuf * n_main, n_chunks):
            gather_desc(j, j % nbuf).wait()
            write(j, j % nbuf)

    return k(table, gids)


def _transpose_out(wide, n_rows):
    blk = 16384
    grid = (n_rows + blk - 1) // blk
    eye = jnp.eye(_EMB_DIM, dtype=jnp.float32)

    def body(e_ref, x_ref, o_ref):
        o_ref[...] = lax.dot_general(
            e_ref[...], x_ref[:, : _EMB_DIM],
            (((1,), (1,)), ((), ())),
            preferred_element_type=jnp.float32,
        )

    return pl.pallas_call(
        body,
        grid=(grid,),
        in_specs=[
            pl.BlockSpec((_EMB_DIM, _EMB_DIM), lambda i: (0, 0)),
            pl.BlockSpec((blk, _WIDE), lambda i: (i, 0)),
        ],
        out_specs=pl.BlockSpec((_EMB_DIM, blk), lambda i: (0, i)),
        out_shape=jax.ShapeDtypeStruct((_EMB_DIM, n_rows), jnp.float32),
    )(eye, wide)


def kernel(ids, adj, layer_idx, emb_table, fc_w, fc_b):
    del adj
    n_nodes = emb_table.shape[0] - 1
    b = ids.shape[0]
    t = _transform_table(emb_table.T, fc_w, fc_b)
    gids = jnp.where(layer_idx > 0, ids, n_nodes).astype(jnp.int32)
    n_chunks = -(-b // (_NW * _CHUNK))
    wide = _gather_rows(t, gids, n_chunks, b)
    return _transpose_out(wide, b).T
